# baseline (speedup 1.0000x reference)
import functools

import jax
import jax.numpy as jnp
from jax import lax
from jax.experimental import pallas as pl
from jax.experimental.pallas import tpu as pltpu
from jax.experimental.pallas import tpu_sc as plsc

N_TOKENS = 8192
D = 2048
E = 64
K = 8

ROW_BLOCK = 1024

NC, NS = 2, 16
NW = NC * NS
ROWS_PER_TILE = N_TOKENS // NW


def _matmul_body(x_ref, w_ref, o_ref):
    o_ref[...] = jnp.dot(x_ref[...], w_ref[...],
                         preferred_element_type=jnp.float32)


def _logits(x, w):
    return pl.pallas_call(
        _matmul_body,
        grid=(N_TOKENS // ROW_BLOCK,),
        in_specs=[
            pl.BlockSpec((ROW_BLOCK, D), lambda i: (i, 0)),
            pl.BlockSpec((D, E), lambda i: (0, 0)),
        ],
        out_specs=pl.BlockSpec((ROW_BLOCK, E), lambda i: (i, 0)),
        out_shape=jax.ShapeDtypeStruct((N_TOKENS, E), jnp.float32),
        compiler_params=pltpu.CompilerParams(
            dimension_semantics=("arbitrary",),
        ),
    )(x, w)


_GATHER_DN = lax.GatherDimensionNumbers(
    offset_dims=(), collapsed_slice_dims=(0,), start_index_map=(0,))


def _take(x, idx):
    return lax.gather(x, idx[:, None], _GATHER_DN, slice_sizes=(1,),
                      mode=lax.GatherScatterMode.PROMISE_IN_BOUNDS)


def _topk_body(logits_hbm, w_hbm, i_hbm, lg_v, wout_v, iout_v, wsem, isem):
    wid = lax.axis_index("s") * NC + lax.axis_index("c")
    base = wid * ROWS_PER_TILE
    pltpu.sync_copy(logits_hbm.at[pl.ds(base, ROWS_PER_TILE)], lg_v)

    lane = lax.iota(jnp.int32, 16)
    lo8 = lane < 8
    rot4 = (lane + 4) % 8
    rot2 = (lane + 2) % 8
    rot1 = (lane + 1) % 8
    kbase = lane * ROWS_PER_TILE

    @plsc.parallel_loop(0, ROWS_PER_TILE, unroll=4)
    def row(r):
        def srt(j, descending):
            k = lg_v[r, pl.ds(j * 16, 16)]
            v = lane + (j * 16)
            return plsc.sort_key_val(k, v, descending=descending)

        k0, v0 = srt(0, True)
        k1, v1 = srt(1, False)
        k2, v2 = srt(2, True)
        k3, v3 = srt(3, False)
        m01k = jnp.where(lo8, k0, k1)
        m01v = jnp.where(lo8, v0, v1)
        m23k = jnp.where(lo8, k2, k3)
        m23v = jnp.where(lo8, v2, v3)
        t01k, t01v = plsc.sort_key_val(m01k, m01v, descending=True)
        t23k, t23v = plsc.sort_key_val(m23k, m23v, descending=False)
        fk_in = jnp.where(lo8, t01k, t23k)
        fv_in = jnp.where(lo8, t01v, t23v)
        fk, fv = plsc.sort_key_val(fk_in, fv_in, descending=True)

        sq = fk * fk
        s = sq + _take(sq, rot4)
        s = s + _take(s, rot2)
        s = s + _take(s, rot1)
        bits = plsc.bitcast(s, jnp.int32)
        y = plsc.bitcast(jnp.int32(0x5F3759DF) - (bits >> 1), jnp.float32)
        half = s * 0.5
        y = y * (1.5 - half * y * y)
        y = y * (1.5 - half * y * y)
        y = y * (1.5 - half * y * y)
        wn = fk * y

        idx = kbase + r
        plsc.store_scatter(wout_v, [idx], wn, mask=lo8)
        plsc.store_scatter(iout_v, [idx], fv, mask=lo8)

    cps = []
    for k in range(K):
        seg = pl.ds(k * ROWS_PER_TILE, ROWS_PER_TILE)
        cps.append(pltpu.make_async_copy(
            wout_v.at[seg], w_hbm.at[k, pl.ds(base, ROWS_PER_TILE)], wsem))
        cps.append(pltpu.make_async_copy(
            iout_v.at[seg], i_hbm.at[k, pl.ds(base, ROWS_PER_TILE)], isem))
    for cp in cps:
        cp.start()
    for cp in cps:
        cp.wait()


def _topk(logits):
    mesh = plsc.VectorSubcoreMesh(core_axis_name="c", subcore_axis_name="s",
                                  num_cores=NC, num_subcores=NS)
    f = pl.kernel(
        _topk_body,
        out_type=(
            jax.ShapeDtypeStruct((K, N_TOKENS), jnp.float32),
            jax.ShapeDtypeStruct((K, N_TOKENS), jnp.int32),
        ),
        mesh=mesh,
        scratch_types=[
            pltpu.VMEM((ROWS_PER_TILE, E), jnp.float32),
            pltpu.VMEM((ROWS_PER_TILE * K,), jnp.float32),
            pltpu.VMEM((ROWS_PER_TILE * K,), jnp.int32),
            pltpu.SemaphoreType.DMA,
            pltpu.SemaphoreType.DMA,
        ],
        compiler_params=pltpu.CompilerParams(needs_layout_passes=False),
    )
    return f(logits)


def _relayout_body(wt_ref, it_ref, w_ref, i_ref):
    w_ref[...] = wt_ref[...].T
    i_ref[...] = it_ref[...].T


def _relayout(w_t, i_t):
    return pl.pallas_call(
        _relayout_body,
        out_shape=(jax.ShapeDtypeStruct((N_TOKENS, K), jnp.float32),
                   jax.ShapeDtypeStruct((N_TOKENS, K), jnp.int32)),
    )(w_t, i_t)


def kernel(x, W_router):
    logits = _logits(x, W_router)
    w_t, i_t = _topk(logits)
    w2, i2 = _relayout(w_t, i_t)
    return (logits, w2, i2)


# speedup vs baseline: 1.2221x; 1.2221x over previous
import functools

import jax
import jax.numpy as jnp
from jax import lax
from jax.experimental import pallas as pl
from jax.experimental.pallas import tpu as pltpu
from jax.experimental.pallas import tpu_sc as plsc

N_TOKENS = 8192
D = 2048
E = 64
K = 8

ROW_BLOCK = 1024

NC, NS = 2, 16
NW = NC * NS
ROWS_PER_TILE = N_TOKENS // NW


def _matmul_body(x_ref, w_ref, o_ref):
    o_ref[...] = jnp.dot(x_ref[...], w_ref[...],
                         preferred_element_type=jnp.float32)


def _logits(x, w):
    return pl.pallas_call(
        _matmul_body,
        grid=(N_TOKENS // ROW_BLOCK,),
        in_specs=[
            pl.BlockSpec((ROW_BLOCK, D), lambda i: (i, 0)),
            pl.BlockSpec((D, E), lambda i: (0, 0)),
        ],
        out_specs=pl.BlockSpec((ROW_BLOCK, E), lambda i: (i, 0)),
        out_shape=jax.ShapeDtypeStruct((N_TOKENS, E), jnp.float32),
        compiler_params=pltpu.CompilerParams(
            dimension_semantics=("arbitrary",),
        ),
    )(x, w)


_GATHER_DN = lax.GatherDimensionNumbers(
    offset_dims=(), collapsed_slice_dims=(0,), start_index_map=(0,))


def _take(x, idx):
    return lax.gather(x, idx[:, None], _GATHER_DN, slice_sizes=(1,),
                      mode=lax.GatherScatterMode.PROMISE_IN_BOUNDS)


def _topk_body(logits_hbm, w_hbm, i_hbm, lg_v, wout_v, iout_v, wsem, isem):
    wid = lax.axis_index("s") * NC + lax.axis_index("c")
    base = wid * ROWS_PER_TILE
    pltpu.sync_copy(logits_hbm.at[pl.ds(base, ROWS_PER_TILE)], lg_v)

    lane = lax.iota(jnp.int32, 16)
    lo8 = lane < 8
    rot4 = (lane + 4) % 8
    rot2 = (lane + 2) % 8
    rot1 = (lane + 1) % 8
    kbase = lane * ROWS_PER_TILE

    @plsc.parallel_loop(0, ROWS_PER_TILE, unroll=4)
    def row(r):
        def srt(j, descending):
            k = lg_v[r, pl.ds(j * 16, 16)]
            v = lane + (j * 16)
            return plsc.sort_key_val(k, v, descending=descending)

        k0, v0 = srt(0, True)
        k1, v1 = srt(1, False)
        k2, v2 = srt(2, True)
        k3, v3 = srt(3, False)
        m01k = jnp.where(lo8, k0, k1)
        m01v = jnp.where(lo8, v0, v1)
        m23k = jnp.where(lo8, k2, k3)
        m23v = jnp.where(lo8, v2, v3)
        t01k, t01v = plsc.sort_key_val(m01k, m01v, descending=True)
        t23k, t23v = plsc.sort_key_val(m23k, m23v, descending=False)
        fk_in = jnp.where(lo8, t01k, t23k)
        fv_in = jnp.where(lo8, t01v, t23v)
        fk, fv = plsc.sort_key_val(fk_in, fv_in, descending=True)

        sq = fk * fk
        s = sq + _take(sq, rot4)
        s = s + _take(s, rot2)
        s = s + _take(s, rot1)
        bits = plsc.bitcast(s, jnp.int32)
        y = plsc.bitcast(jnp.int32(0x5F3759DF) - (bits >> 1), jnp.float32)
        half = s * 0.5
        y = y * (1.5 - half * y * y)
        y = y * (1.5 - half * y * y)
        y = y * (1.5 - half * y * y)
        wn = fk * y

        idx = kbase + r
        plsc.store_scatter(wout_v, [idx], wn, mask=lo8)
        plsc.store_scatter(iout_v, [idx], fv, mask=lo8)

    cps = []
    for k in range(K):
        seg = pl.ds(k * ROWS_PER_TILE, ROWS_PER_TILE)
        cps.append(pltpu.make_async_copy(
            wout_v.at[seg], w_hbm.at[k, pl.ds(base, ROWS_PER_TILE)], wsem))
        cps.append(pltpu.make_async_copy(
            iout_v.at[seg], i_hbm.at[k, pl.ds(base, ROWS_PER_TILE)], isem))
    for cp in cps:
        cp.start()
    for cp in cps:
        cp.wait()


def _topk(logits):
    mesh = plsc.VectorSubcoreMesh(core_axis_name="c", subcore_axis_name="s",
                                  num_cores=NC, num_subcores=NS)
    f = pl.kernel(
        _topk_body,
        out_type=(
            jax.ShapeDtypeStruct((K, N_TOKENS), jnp.float32),
            jax.ShapeDtypeStruct((K, N_TOKENS), jnp.int32),
        ),
        mesh=mesh,
        scratch_types=[
            pltpu.VMEM((ROWS_PER_TILE, E), jnp.float32),
            pltpu.VMEM((ROWS_PER_TILE * K,), jnp.float32),
            pltpu.VMEM((ROWS_PER_TILE * K,), jnp.int32),
            pltpu.SemaphoreType.DMA,
            pltpu.SemaphoreType.DMA,
        ],
        compiler_params=pltpu.CompilerParams(needs_layout_passes=False),
    )
    return f(logits)


def _relayout_body(wt_ref, it_ref, w_ref, i_ref):
    w_ref[...] = wt_ref[...].T
    i_ref[...] = it_ref[...].T


def _relayout(w_t, i_t):
    return pl.pallas_call(
        _relayout_body,
        out_shape=(jax.ShapeDtypeStruct((N_TOKENS, K), jnp.float32),
                   jax.ShapeDtypeStruct((N_TOKENS, K), jnp.int32)),
    )(w_t, i_t)


def kernel(x, W_router):
    logits = _logits(x, W_router)
    w_t, i_t = _topk(logits)
    return (logits, w_t.T, i_t.T)
